# baseline (speedup 1.0000x reference)
import functools

import jax
import jax.numpy as jnp
from jax import lax
from jax.experimental import pallas as pl
from jax.experimental.pallas import tpu as pltpu
from jax.experimental.pallas import tpu_sc as plsc

VOCAB = 250002
B = 64
L = 512
D = 1024
NTILES = 32
VT = 1954
VPAD = VT * 128
PHYS = (B // 8) * VT * 1024
TILE_SPAN = PHYS // NTILES
ZCH = TILE_SPAN // 32


def _row_pass(ha, hb, wa, wb, bias, idr, idc, vals_ref, gidx_ref, rr, t):
    tw = (jnp.sum(ha * wa, axis=1, keepdims=True)
          + jnp.sum(hb * wb, axis=1, keepdims=True) + bias)
    eq = idc == idr
    cand = jnp.where(eq, tw, -jnp.inf)
    segmax = jnp.max(cand, axis=0, keepdims=True)
    v = jnp.where(idr < 4, 0.0, segmax)
    r = 8 * t + rr
    off = (((r // 8) * VT + (idr >> 7)) * 1024
           + (r % 8) * 128 + (idr & 127))
    half = rr % 2
    vals_ref[rr // 2, 4 * half:4 * half + 4, :] = v.reshape(4, 128)
    gidx_ref[rr // 2, 4 * half:4 * half + 4, :] = off.reshape(4, 128)


def _tc_body(ha_ref, hb_ref, w_ref, b_ref, ids_ref, idst_ref,
             vals_ref, gidx_ref):
    t = pl.program_id(0)
    wa = w_ref[:, :D // 2]
    wb = w_ref[:, D // 2:]
    bias = b_ref[0, 0]
    for rr in range(8):
        idr = ids_ref[rr:rr + 1, :]
        idc = idst_ref[:, rr, :]
        _row_pass(ha_ref[rr], hb_ref[rr], wa, wb, bias,
                  idr, idc, vals_ref, gidx_ref, rr, t)


def _tc_weights(hidden_states, W, b2, input_ids, ids_t):
    return pl.pallas_call(
        _tc_body,
        grid=(B // 8,),
        in_specs=[
            pl.BlockSpec((8, L, D // 2), lambda t: (t, 0, 0)),
            pl.BlockSpec((8, L, D // 2), lambda t: (t, 0, 1)),
            pl.BlockSpec((1, D), lambda t: (0, 0)),
            pl.BlockSpec((1, 1), lambda t: (0, 0)),
            pl.BlockSpec((8, L), lambda t: (t, 0)),
            pl.BlockSpec((L, 8, 1), lambda t: (0, t, 0)),
        ],
        out_specs=[
            pl.BlockSpec((4, 8, 128), lambda t: (t, 0, 0)),
            pl.BlockSpec((4, 8, 128), lambda t: (t, 0, 0)),
        ],
        out_shape=[
            jax.ShapeDtypeStruct((NTILES, 8, 128), jnp.float32),
            jax.ShapeDtypeStruct((NTILES, 8, 128), jnp.int32),
        ],
        compiler_params=pltpu.CompilerParams(
            dimension_semantics=("arbitrary",)),
    )(hidden_states, hidden_states, W, b2, input_ids, ids_t)


def _sc_scatter_body(gidx_hbm, vals_hbm, zsrc_hbm, out_hbm,
                     idx_v, val_v, zbuf, sem_in, sem_z, sem_sc):
    c = lax.axis_index("c")
    s = lax.axis_index("s")
    wid = c * 16 + s
    cp_z = pltpu.async_copy(zsrc_hbm, zbuf, sem_in)
    cp_i = pltpu.async_copy(gidx_hbm.at[wid], idx_v, sem_in)
    cp_v = pltpu.async_copy(vals_hbm.at[wid], val_v, sem_in)

    base = wid * TILE_SPAN

    cp_z.wait()
    zws = []
    for k in range(32):
        off = pl.multiple_of(base + k * ZCH, 8)
        zws.append(pltpu.async_copy(
            zbuf, out_hbm.at[pl.ds(off, ZCH)], sem_z))
    for cp in zws:
        cp.wait()
    plsc.subcore_barrier()

    cp_i.wait()
    cp_v.wait()
    scs = []
    for j in range(8):
        scs.append(pltpu.async_copy(
            val_v.at[j], out_hbm.at[idx_v.at[j]], sem_sc))
    for cp in scs:
        cp.wait()


def _sc_scatter(gidx3, vals3, zsrc):
    mesh = plsc.VectorSubcoreMesh(core_axis_name="c", subcore_axis_name="s")
    fn = functools.partial(
        pl.kernel,
        out_type=jax.ShapeDtypeStruct((PHYS,), jnp.float32),
        mesh=mesh,
        scratch_types=[
            pltpu.VMEM((8, 128), jnp.int32),
            pltpu.VMEM((8, 128), jnp.float32),
            pltpu.VMEM((ZCH,), jnp.float32),
            pltpu.SemaphoreType.DMA,
            pltpu.SemaphoreType.DMA,
            pltpu.SemaphoreType.DMA,
        ],
    )(_sc_scatter_body)
    return fn(gidx3, vals3, zsrc)


def kernel(hidden_states, W, b, input_ids):
    b2 = b.reshape(1, 1)
    ids_t = input_ids.T[:, :, None]
    vals3, gidx3 = _tc_weights(hidden_states, W, b2, input_ids, ids_t)
    zsrc = jnp.zeros((ZCH,), jnp.float32)
    flat = _sc_scatter(gidx3, vals3, zsrc)
    out = (flat.reshape(B // 8, VT, 8, 128)
           .transpose(0, 2, 1, 3)
           .reshape(B, VPAD)[:, :VOCAB])
    return out


# speedup vs baseline: 1.3701x; 1.3701x over previous
import functools

import jax
import jax.numpy as jnp
from jax import lax
from jax.experimental import pallas as pl
from jax.experimental.pallas import tpu as pltpu
from jax.experimental.pallas import tpu_sc as plsc

VOCAB = 250002
B = 64
L = 512
D = 1024
NTILES = 32
VT = 1954
VPAD = VT * 128
PHYS = (B // 8) * VT * 1024
TILE_SPAN = PHYS // NTILES
ZCH = TILE_SPAN // 32


def _row_pass(ha, hb, wa, wb, bias, idr, idc, vals_ref, gidx_ref, rr, t):
    tw = (jnp.sum(ha * wa, axis=1, keepdims=True)
          + jnp.sum(hb * wb, axis=1, keepdims=True) + bias)
    eq = idc == idr
    cand = jnp.where(eq, tw, -jnp.inf)
    segmax = jnp.max(cand, axis=0, keepdims=True)
    v = jnp.where(idr < 4, 0.0, segmax)
    r = 4 * t + rr
    off = (((r // 8) * VT + (idr >> 7)) * 1024
           + (r % 8) * 128 + (idr & 127))
    half = rr % 2
    vals_ref[rr // 2, 4 * half:4 * half + 4, :] = v.reshape(4, 128)
    gidx_ref[rr // 2, 4 * half:4 * half + 4, :] = off.reshape(4, 128)


def _tc_body(ha_ref, hb_ref, w_ref, b_ref,
             idr0_ref, idr1_ref, idr2_ref, idr3_ref,
             idc0_ref, idc1_ref, idc2_ref, idc3_ref,
             vals_ref, gidx_ref):
    t = pl.program_id(0)
    wa = w_ref[:, :D // 2]
    wb = w_ref[:, D // 2:]
    bias = b_ref[0, 0]
    idrs = (idr0_ref, idr1_ref, idr2_ref, idr3_ref)
    idcs = (idc0_ref, idc1_ref, idc2_ref, idc3_ref)
    for rr in range(4):
        _row_pass(ha_ref[rr], hb_ref[rr], wa, wb, bias,
                  idrs[rr][0], idcs[rr][0], vals_ref, gidx_ref, rr, t)


def _tc_weights(hidden_states, W, b2, ids_row, ids_col):
    row_spec = [pl.BlockSpec((1, 1, L), (lambda rr: (lambda t: (4 * t + rr, 0, 0)))(i))
                for i in range(4)]
    col_spec = [pl.BlockSpec((1, L, 1), (lambda rr: (lambda t: (4 * t + rr, 0, 0)))(i))
                for i in range(4)]
    return pl.pallas_call(
        _tc_body,
        grid=(B // 4,),
        in_specs=[
            pl.BlockSpec((4, L, D // 2), lambda t: (t, 0, 0)),
            pl.BlockSpec((4, L, D // 2), lambda t: (t, 0, 1)),
            pl.BlockSpec((1, D), lambda t: (0, 0)),
            pl.BlockSpec((1, 1), lambda t: (0, 0)),
            *row_spec,
            *col_spec,
        ],
        out_specs=[
            pl.BlockSpec((2, 8, 128), lambda t: (t, 0, 0)),
            pl.BlockSpec((2, 8, 128), lambda t: (t, 0, 0)),
        ],
        out_shape=[
            jax.ShapeDtypeStruct((NTILES, 8, 128), jnp.float32),
            jax.ShapeDtypeStruct((NTILES, 8, 128), jnp.int32),
        ],
        compiler_params=pltpu.CompilerParams(
            dimension_semantics=("arbitrary",)),
    )(hidden_states, hidden_states, W, b2,
      ids_row, ids_row, ids_row, ids_row,
      ids_col, ids_col, ids_col, ids_col)


def _sc_scatter_body(gidx_hbm, vals_hbm, zsrc_hbm, out_hbm,
                     idx_v, val_v, zbuf, sem_in, sem_z, sem_sc):
    c = lax.axis_index("c")
    s = lax.axis_index("s")
    wid = c * 16 + s
    cp_z = pltpu.async_copy(zsrc_hbm, zbuf, sem_in)
    cp_i = pltpu.async_copy(gidx_hbm.at[wid], idx_v, sem_in)
    cp_v = pltpu.async_copy(vals_hbm.at[wid], val_v, sem_in)

    base = wid * TILE_SPAN

    cp_z.wait()
    zws = []
    for k in range(32):
        off = pl.multiple_of(base + k * ZCH, 8)
        zws.append(pltpu.async_copy(
            zbuf, out_hbm.at[pl.ds(off, ZCH)], sem_z))
    for cp in zws:
        cp.wait()
    plsc.subcore_barrier()

    cp_i.wait()
    cp_v.wait()
    scs = []
    for j in range(8):
        scs.append(pltpu.async_copy(
            val_v.at[j], out_hbm.at[idx_v.at[j]], sem_sc))
    for cp in scs:
        cp.wait()


def _sc_scatter(gidx3, vals3, zsrc):
    mesh = plsc.VectorSubcoreMesh(core_axis_name="c", subcore_axis_name="s")
    fn = functools.partial(
        pl.kernel,
        out_type=jax.ShapeDtypeStruct((PHYS,), jnp.float32),
        mesh=mesh,
        scratch_types=[
            pltpu.VMEM((8, 128), jnp.int32),
            pltpu.VMEM((8, 128), jnp.float32),
            pltpu.VMEM((ZCH,), jnp.float32),
            pltpu.SemaphoreType.DMA,
            pltpu.SemaphoreType.DMA,
            pltpu.SemaphoreType.DMA,
        ],
    )(_sc_scatter_body)
    return fn(gidx3, vals3, zsrc)


def kernel(hidden_states, W, b, input_ids):
    b2 = b.reshape(1, 1)
    ids_row = input_ids[:, None, :]
    ids_col = input_ids[:, :, None]
    vals3, gidx3 = _tc_weights(hidden_states, W, b2, ids_row, ids_col)
    zsrc = jnp.zeros((ZCH,), jnp.float32)
    flat = _sc_scatter(gidx3, vals3, zsrc)
    out = (flat.reshape(B // 8, VT, 8, 128)
           .transpose(0, 2, 1, 3)
           .reshape(B, VPAD)[:, :VOCAB])
    return out


# speedup vs baseline: 1.5050x; 1.0985x over previous
import functools

import jax
import jax.numpy as jnp
from jax import lax
from jax.experimental import pallas as pl
from jax.experimental.pallas import tpu as pltpu
from jax.experimental.pallas import tpu_sc as plsc

VOCAB = 250002
B = 64
L = 512
D = 1024
NTILES = 32
VT = 1954
VPAD = VT * 128
PHYS = (B // 8) * VT * 1024
TILE_SPAN = PHYS // NTILES
ZCH = TILE_SPAN // 32


def _row_pass(ha, hb, wa, wb, bias, idr, vals_ref, gidx_ref, rr, t):
    tw = (jnp.sum(ha * wa, axis=1, keepdims=True)
          + jnp.sum(hb * wb, axis=1, keepdims=True) + bias)
    idc = jnp.swapaxes(idr, 0, 1)
    eq = idc == idr
    cand = jnp.where(eq, tw, -jnp.inf)
    segmax = jnp.max(cand, axis=0, keepdims=True)
    v = jnp.where(idr < 4, 0.0, segmax)
    r = 4 * t + rr
    off = (((r // 8) * VT + (idr >> 7)) * 1024
           + (r % 8) * 128 + (idr & 127))
    half = rr % 2
    vals_ref[rr // 2, 4 * half:4 * half + 4, :] = v.reshape(4, 128)
    gidx_ref[rr // 2, 4 * half:4 * half + 4, :] = off.reshape(4, 128)


def _tc_body(ha_ref, hb_ref, w_ref, b_ref,
             idr0_ref, idr1_ref, idr2_ref, idr3_ref,
             vals_ref, gidx_ref):
    t = pl.program_id(0)
    wa = w_ref[:, :D // 2]
    wb = w_ref[:, D // 2:]
    bias = b_ref[0, 0]
    idrs = (idr0_ref, idr1_ref, idr2_ref, idr3_ref)
    for rr in range(4):
        _row_pass(ha_ref[rr], hb_ref[rr], wa, wb, bias,
                  idrs[rr][0], vals_ref, gidx_ref, rr, t)


def _tc_weights(hidden_states, W, b2, ids_row):
    row_spec = [pl.BlockSpec((1, 1, L), (lambda rr: (lambda t: (4 * t + rr, 0, 0)))(i))
                for i in range(4)]
    return pl.pallas_call(
        _tc_body,
        grid=(B // 4,),
        in_specs=[
            pl.BlockSpec((4, L, D // 2), lambda t: (t, 0, 0)),
            pl.BlockSpec((4, L, D // 2), lambda t: (t, 0, 1)),
            pl.BlockSpec((1, D), lambda t: (0, 0)),
            pl.BlockSpec((1, 1), lambda t: (0, 0)),
            *row_spec,
        ],
        out_specs=[
            pl.BlockSpec((2, 8, 128), lambda t: (t, 0, 0)),
            pl.BlockSpec((2, 8, 128), lambda t: (t, 0, 0)),
        ],
        out_shape=[
            jax.ShapeDtypeStruct((NTILES, 8, 128), jnp.float32),
            jax.ShapeDtypeStruct((NTILES, 8, 128), jnp.int32),
        ],
        compiler_params=pltpu.CompilerParams(
            dimension_semantics=("arbitrary",)),
    )(hidden_states, hidden_states, W, b2,
      ids_row, ids_row, ids_row, ids_row)


def _sc_scatter_body(gidx_hbm, vals_hbm, zsrc_hbm, out_hbm,
                     idx_v, val_v, zbuf, sem_in, sem_z, sem_sc):
    c = lax.axis_index("c")
    s = lax.axis_index("s")
    wid = c * 16 + s
    cp_z = pltpu.async_copy(zsrc_hbm, zbuf, sem_in)
    cp_i = pltpu.async_copy(gidx_hbm.at[wid], idx_v, sem_in)
    cp_v = pltpu.async_copy(vals_hbm.at[wid], val_v, sem_in)

    base = wid * TILE_SPAN

    cp_z.wait()
    zws = []
    for k in range(32):
        off = pl.multiple_of(base + k * ZCH, 8)
        zws.append(pltpu.async_copy(
            zbuf, out_hbm.at[pl.ds(off, ZCH)], sem_z))
    for cp in zws:
        cp.wait()
    plsc.subcore_barrier()

    cp_i.wait()
    cp_v.wait()
    scs = []
    for j in range(8):
        scs.append(pltpu.async_copy(
            val_v.at[j], out_hbm.at[idx_v.at[j]], sem_sc))
    for cp in scs:
        cp.wait()


def _sc_scatter(gidx3, vals3, zsrc):
    mesh = plsc.VectorSubcoreMesh(core_axis_name="c", subcore_axis_name="s")
    fn = functools.partial(
        pl.kernel,
        out_type=jax.ShapeDtypeStruct((PHYS,), jnp.float32),
        mesh=mesh,
        scratch_types=[
            pltpu.VMEM((8, 128), jnp.int32),
            pltpu.VMEM((8, 128), jnp.float32),
            pltpu.VMEM((ZCH,), jnp.float32),
            pltpu.SemaphoreType.DMA,
            pltpu.SemaphoreType.DMA,
            pltpu.SemaphoreType.DMA,
        ],
    )(_sc_scatter_body)
    return fn(gidx3, vals3, zsrc)


def kernel(hidden_states, W, b, input_ids):
    b2 = b.reshape(1, 1)
    ids_row = input_ids[:, None, :]
    vals3, gidx3 = _tc_weights(hidden_states, W, b2, ids_row)
    zsrc = jnp.zeros((ZCH,), jnp.float32)
    flat = _sc_scatter(gidx3, vals3, zsrc)
    out = (flat.reshape(B // 8, VT, 8, 128)
           .transpose(0, 2, 1, 3)
           .reshape(B, VPAD)[:, :VOCAB])
    return out
